# baseline (speedup 1.0000x reference)
import jax
import jax.numpy as jnp
from jax.experimental import pallas as pl
from jax.experimental.pallas import tpu as pltpu

_B = 512
_F = 28 * 28 * 3
_BLK = 64


def _body(p_ref, t_ref, out_ref, acc_ref):
    i = pl.program_id(0)

    @pl.when(i == 0)
    def _init():
        acc_ref[0] = 0.0
        acc_ref[1] = 0.0

    t = t_ref[...]
    p = p_ref[...]
    ch = jax.lax.broadcasted_iota(jnp.int32, t.shape, 1) % 3
    is0 = ch == 0
    t0m = jnp.where(is0, t, 0.0)
    t0b = t0m + pltpu.roll(t0m, 1, 1) + pltpu.roll(t0m, 2, 1)
    coo = t0b > 0.5
    cw = jnp.where(ch == 1, 5.0, 1.0)
    w = jnp.where(coo, cw, 0.0) + jnp.where((t0b < 0.5) & is0, 0.5, 0.0)
    d = p - t
    s = jnp.sum(w * d * d, axis=1, keepdims=True)
    tmax = jnp.max(t, axis=1, keepdims=True)
    sel = tmax > 0.5
    acc_ref[0] += jnp.sum(jnp.where(sel, s, 0.0))
    acc_ref[1] += jnp.sum(sel.astype(jnp.float32))

    @pl.when(i == pl.num_programs(0) - 1)
    def _fin():
        out_ref[0] = acc_ref[0] / (28.0 * acc_ref[1])


def kernel(pred_tensor, target_tensor):
    p = pred_tensor.reshape(_B, _F)
    t = target_tensor.reshape(_B, _F)
    out = pl.pallas_call(
        _body,
        grid=(_B // _BLK,),
        in_specs=[
            pl.BlockSpec((_BLK, _F), lambda i: (i, 0)),
            pl.BlockSpec((_BLK, _F), lambda i: (i, 0)),
        ],
        out_specs=pl.BlockSpec(memory_space=pltpu.SMEM),
        out_shape=jax.ShapeDtypeStruct((1,), jnp.float32),
        scratch_shapes=[pltpu.SMEM((2,), jnp.float32)],
    )(p, t)
    return out[0]
